# baseline (speedup 1.0000x reference)
import jax
import jax.numpy as jnp
from jax import lax
from jax.experimental import pallas as pl
from jax.experimental.pallas import tpu as pltpu

_K = 3
_ITERS = 5
_ROWS = 2048
_COLS = 128
_LANES = 3 * _COLS


def _kmeans_body(dep_ref, c_ref, v_ref, o_ref):
    v = v_ref[...]
    f32 = jnp.float32

    rj = lax.broadcasted_iota(jnp.int32, (_LANES, _COLS), 0)
    cp = lax.broadcasted_iota(jnp.int32, (_LANES, _COLS), 1)
    px = (rj == 3 * cp).astype(f32)
    py = (rj == 3 * cp + 1).astype(f32)
    pz = (rj == 3 * cp + 2).astype(f32)
    x = jnp.dot(v, px, precision=lax.Precision.HIGHEST)
    y = jnp.dot(v, py, precision=lax.Precision.HIGHEST)
    z = jnp.dot(v, pz, precision=lax.Precision.HIGHEST)

    nn = f32(_ROWS * _COLS)
    sx_t = jnp.sum(x)
    sy_t = jnp.sum(y)
    sz_t = jnp.sum(z)

    def lin_forms(c):
        c0x, c0y, c0z, c1x, c1y, c1z, c2x, c2y, c2z = c
        q0 = c0x * c0x + c0y * c0y + c0z * c0z
        q1 = c1x * c1x + c1y * c1y + c1z * c1z
        q2 = c2x * c2x + c2y * c2y + c2z * c2z
        g0 = x * (-2.0 * c0x) + y * (-2.0 * c0y) + z * (-2.0 * c0z) + q0
        g1 = x * (-2.0 * c1x) + y * (-2.0 * c1y) + z * (-2.0 * c1z) + q1
        g2 = x * (-2.0 * c2x) + y * (-2.0 * c2y) + z * (-2.0 * c2z) + q2
        return g0, g1, g2

    def masks_from(c):
        g0, g1, g2 = lin_forms(c)
        lt1 = g1 < g0
        sel2 = g2 < jnp.minimum(g0, g1)
        n2 = jnp.logical_not(sel2)
        sel1 = jnp.logical_and(lt1, n2)
        sel0 = jnp.logical_and(jnp.logical_not(lt1), n2)
        return sel0, sel1

    zero = f32(0.0)
    c = tuple(c_ref[i, j] for i in range(_K) for j in range(3))
    for _ in range(_ITERS - 1):
        sel0, sel1 = masks_from(c)
        f0 = jnp.where(sel0, 1.0, zero)
        f1 = jnp.where(sel1, 1.0, zero)
        n0 = jnp.sum(f0)
        n1 = jnp.sum(f1)
        n2 = nn - n0 - n1
        sx0 = jnp.sum(jnp.where(sel0, x, zero))
        sy0 = jnp.sum(jnp.where(sel0, y, zero))
        sz0 = jnp.sum(jnp.where(sel0, z, zero))
        sx1 = jnp.sum(jnp.where(sel1, x, zero))
        sy1 = jnp.sum(jnp.where(sel1, y, zero))
        sz1 = jnp.sum(jnp.where(sel1, z, zero))
        c = (sx0 / n0, sy0 / n0, sz0 / n0,
             sx1 / n1, sy1 / n1, sz1 / n1,
             (sx_t - sx0 - sx1) / n2,
             (sy_t - sy0 - sy1) / n2,
             (sz_t - sz0 - sz1) / n2)

    sel0, _ = masks_from(c)
    f0 = jnp.where(sel0, 1.0, zero)
    pi = lax.broadcasted_iota(jnp.int32, (_COLS, _LANES), 0)
    ji = lax.broadcasted_iota(jnp.int32, (_COLS, _LANES), 1)
    e = jnp.logical_and(ji >= 3 * pi, ji < 3 * pi + 3).astype(f32)
    mi = jnp.dot(f0, e, precision=lax.Precision.HIGHEST)
    base = dep_ref[0]
    o_ref[...] = (1.0 - mi) * base


def kernel(data, img_shape):
    data = data.reshape((-1, 3))
    n = data.shape[0]
    init_idx = jax.random.randint(jax.random.key(42), (3,), 0, n)
    centers = jnp.take(data, init_idx, axis=0)
    dep = ((jnp.asarray(img_shape[0]) + jnp.asarray(img_shape[1])
            + jnp.asarray(img_shape[2])) * 0).astype(data.dtype).reshape(1)
    v = data.reshape(_ROWS, _LANES)

    out = pl.pallas_call(
        _kmeans_body,
        in_specs=[
            pl.BlockSpec(memory_space=pltpu.SMEM),
            pl.BlockSpec(memory_space=pltpu.SMEM),
            pl.BlockSpec(memory_space=pltpu.VMEM),
        ],
        out_specs=pl.BlockSpec(memory_space=pltpu.VMEM),
        out_shape=jax.ShapeDtypeStruct((_ROWS, _LANES), jnp.float32),
    )(dep, centers, v)

    return out.reshape(n, 1, 3)
